# baseline (speedup 1.0000x reference)
import jax
import jax.numpy as jnp
from jax import lax
from jax.experimental import pallas as pl
from jax.experimental.pallas import tpu as pltpu
from jax.experimental.pallas import tpu_sc as plsc

_HID = 2048
_B = 2
_S = 4096
_EPS = 1e-6
_L = 16
_NC = 2
_NS = 16
_NW = _NC * _NS
_N = _B * _S
_RPW = _N // _NW
_CHUNK = 16
_NCHUNK = _RPW // _CHUNK
_SLICES = _HID // _L


def _body(ids_hbm, w_hbm, table_hbm, out_hbm, idx_v, w_v, buf_v, sem0, sem1):
    wid = lax.axis_index("s") * _NC + lax.axis_index("c")
    base = wid * _RPW
    pltpu.sync_copy(ids_hbm.at[pl.ds(base, _RPW)], idx_v)
    pltpu.sync_copy(w_hbm, w_v)
    sems = (sem0, sem1)
    iota = lax.iota(jnp.int32, _L)

    def start_gather(c, slot):
        return pltpu.async_copy(
            table_hbm.at[idx_v.at[pl.ds(c * _CHUNK, _CHUNK)]],
            buf_v.at[slot],
            sems[slot],
        )

    def process(slot):
        def p_a(j, accs):
            o = j * (2 * _L)
            new = []
            for r in range(_CHUNK):
                x0 = buf_v[slot, r, pl.ds(o, _L)]
                x1 = buf_v[slot, r, pl.ds(o + _L, _L)]
                new.append(accs[r] + (x0 * x0 + x1 * x1))
            return tuple(new)

        accs = lax.fori_loop(
            0, _SLICES // 2, p_a,
            tuple(jnp.zeros((_L,), jnp.float32) for _ in range(_CHUNK)))

        vecs = list(accs)
        stage = 0
        while len(vecs) > 1:
            g = 1 << stage
            sel = ((iota >> stage) & 1) == 0
            perm = jnp.bitwise_xor(iota, g)
            nxt = []
            for k in range(len(vecs) // 2):
                a, b = vecs[2 * k], vecs[2 * k + 1]
                ap = a.at[perm].get(mode="promise_in_bounds")
                bp = b.at[perm].get(mode="promise_in_bounds")
                nxt.append(jnp.where(sel, a, b) + jnp.where(sel, ap, bp))
            vecs = nxt
            stage += 1
        s = vecs[0]
        vv = s * (1.0 / _HID) + _EPS
        bits = lax.bitcast_convert_type(vv, jnp.int32)
        bits = jnp.full((_L,), 0x5F3759DF, jnp.int32) - \
            lax.shift_right_logical(bits, 1)
        y = lax.bitcast_convert_type(bits, jnp.float32)
        for _ in range(4):
            y = y * (1.5 - (0.5 * vv) * (y * y))
        ysplat = [
            y.at[jnp.full((_L,), r, jnp.int32)].get(mode="promise_in_bounds")
            for r in range(_CHUNK)
        ]

        def p_c(j, carry):
            o = j * (2 * _L)
            sl0 = pl.ds(o, _L)
            sl1 = pl.ds(o + _L, _L)
            w0 = w_v[sl0]
            w1 = w_v[sl1]
            for r in range(_CHUNK):
                buf_v[slot, r, sl0] = buf_v[slot, r, sl0] * ysplat[r] * w0
                buf_v[slot, r, sl1] = buf_v[slot, r, sl1] * ysplat[r] * w1
            return carry

        lax.fori_loop(0, _SLICES // 2, p_c, 0)

    copies = [start_gather(0, 0), start_gather(1, 1)]
    for c in range(_NCHUNK):
        slot = c % 2
        copies[slot].wait()
        process(slot)
        pltpu.sync_copy(buf_v.at[slot],
                        out_hbm.at[pl.ds(base + c * _CHUNK, _CHUNK)])
        if c + 2 < _NCHUNK:
            copies[slot] = start_gather(c + 2, slot)


def kernel(input_ids, embed_table, norm_weight):
    ids = input_ids.reshape(-1).astype(jnp.int32)
    mesh = plsc.VectorSubcoreMesh(core_axis_name="c", subcore_axis_name="s")
    f = pl.kernel(
        _body,
        mesh=mesh,
        out_type=jax.ShapeDtypeStruct((_N, _HID), jnp.float32),
        scratch_types=[
            pltpu.VMEM((_RPW,), jnp.int32),
            pltpu.VMEM((_HID,), jnp.float32),
            pltpu.VMEM((2, _CHUNK, _HID), jnp.float32),
            pltpu.SemaphoreType.DMA,
            pltpu.SemaphoreType.DMA,
        ],
    )
    out = f(ids, norm_weight.astype(jnp.float32), embed_table)
    return out.reshape(_B, _S, _HID)


# speedup vs baseline: 2.2745x; 2.2745x over previous
import jax
import jax.numpy as jnp
from jax import lax
from jax.experimental import pallas as pl
from jax.experimental.pallas import tpu as pltpu
from jax.experimental.pallas import tpu_sc as plsc

_HID = 2048
_B = 2
_S = 4096
_EPS = 1e-6
_L = 16
_NC = 2
_NS = 16
_NW = _NC * _NS
_N = _B * _S
_RPW = _N // _NW
_CHUNK = 16
_NCHUNK = _RPW // _CHUNK
_SLICES = _HID // _L


def _body(ids_hbm, w_hbm, table_hbm, out_hbm, idx_v, w_v, buf_v, sem0, sem1):
    wid = lax.axis_index("s") * _NC + lax.axis_index("c")
    base = wid * _RPW
    pltpu.sync_copy(ids_hbm.at[pl.ds(base, _RPW)], idx_v)
    pltpu.sync_copy(w_hbm, w_v)
    sems = (sem0, sem1)
    iota = lax.iota(jnp.int32, _L)

    def gather_desc(c, slot):
        return pltpu.make_async_copy(
            table_hbm.at[idx_v.at[pl.ds(c * _CHUNK, _CHUNK)]],
            buf_v.at[slot],
            sems[slot],
        )

    def process(slot):
        def p_a(j, accs):
            o = j * _L
            new = []
            for r in range(_CHUNK):
                x = buf_v[slot, r, pl.ds(o, _L)]
                new.append(accs[r] + x * x)
            return tuple(new)

        accs = lax.fori_loop(
            0, _SLICES, p_a,
            tuple(jnp.zeros((_L,), jnp.float32) for _ in range(_CHUNK)))

        vecs = list(accs)
        stage = 0
        while len(vecs) > 1:
            g = 1 << stage
            sel = ((iota >> stage) & 1) == 0
            perm = jnp.bitwise_xor(iota, g)
            nxt = []
            for k in range(len(vecs) // 2):
                a, b = vecs[2 * k], vecs[2 * k + 1]
                ap = a.at[perm].get(mode="promise_in_bounds")
                bp = b.at[perm].get(mode="promise_in_bounds")
                nxt.append(jnp.where(sel, a, b) + jnp.where(sel, ap, bp))
            vecs = nxt
            stage += 1
        s = vecs[0]
        vv = s * (1.0 / _HID) + _EPS
        bits = lax.bitcast_convert_type(vv, jnp.int32)
        bits = jnp.full((_L,), 0x5F3759DF, jnp.int32) - \
            lax.shift_right_logical(bits, 1)
        y = lax.bitcast_convert_type(bits, jnp.float32)
        for _ in range(4):
            y = y * (1.5 - (0.5 * vv) * (y * y))
        ysplat = [
            y.at[jnp.full((_L,), r, jnp.int32)].get(mode="promise_in_bounds")
            for r in range(_CHUNK)
        ]

        def p_c(j, carry):
            o = j * _L
            sl = pl.ds(o, _L)
            w = w_v[sl]
            for r in range(_CHUNK):
                buf_v[slot, r, sl] = buf_v[slot, r, sl] * ysplat[r] * w
            return carry

        lax.fori_loop(0, _SLICES, p_c, 0)

    gather_desc(0, 0).start()
    gather_desc(1, 1).start()

    def pair_body(p, carry):
        for slot in range(2):
            c = 2 * p + slot
            gather_desc(c, slot).wait()
            process(slot)
            pltpu.sync_copy(buf_v.at[slot],
                            out_hbm.at[pl.ds(base + c * _CHUNK, _CHUNK)])

            @pl.when(c + 2 < _NCHUNK)
            def _():
                gather_desc(c + 2, slot).start()
        return carry

    lax.fori_loop(0, _NCHUNK // 2, pair_body, 0)


def kernel(input_ids, embed_table, norm_weight):
    ids = input_ids.reshape(-1).astype(jnp.int32)
    mesh = plsc.VectorSubcoreMesh(core_axis_name="c", subcore_axis_name="s")
    f = pl.kernel(
        _body,
        mesh=mesh,
        out_type=jax.ShapeDtypeStruct((_N, _HID), jnp.float32),
        scratch_types=[
            pltpu.VMEM((_RPW,), jnp.int32),
            pltpu.VMEM((_HID,), jnp.float32),
            pltpu.VMEM((2, _CHUNK, _HID), jnp.float32),
            pltpu.SemaphoreType.DMA,
            pltpu.SemaphoreType.DMA,
        ],
    )
    out = f(ids, norm_weight.astype(jnp.float32), embed_table)
    return out.reshape(_B, _S, _HID)
